# Initial kernel scaffold; baseline (speedup 1.0000x reference)
import functools

import jax
import jax.numpy as jnp
from jax import lax
from jax.experimental import pallas as pl
from jax.experimental.pallas import tpu as pltpu
from jax.experimental.pallas import tpu_sc as plsc

_CHUNK = 80


def _make_sc_gather(n_rows, emb, n_workers):
    rows_per_w = n_rows // n_workers
    n_chunks = rows_per_w // _CHUNK
    assert rows_per_w % _CHUNK == 0

    mesh = plsc.VectorSubcoreMesh(core_axis_name="c", subcore_axis_name="s")

    @functools.partial(
        pl.kernel,
        mesh=mesh,
        out_type=jax.ShapeDtypeStruct((n_rows, emb), jnp.float32),
        scratch_types=[
            pltpu.VMEM((n_chunks, _CHUNK), jnp.int32),
            pltpu.VMEM((rows_per_w, emb), jnp.float32),
            pltpu.SemaphoreType.DMA,
        ],
    )
    def sc_gather(idx_hbm, table_hbm, out_hbm, idx_v, rows_v, sem):
        nc = 2
        wid = lax.axis_index("s") * nc + lax.axis_index("c")
        pltpu.sync_copy(idx_hbm.at[pl.ds(wid * n_chunks, n_chunks)], idx_v)
        copies = []
        for j in range(n_chunks):
            copies.append(
                pltpu.async_copy(
                    table_hbm.at[idx_v.at[j]],
                    rows_v.at[pl.ds(j * _CHUNK, _CHUNK)],
                    sem,
                )
            )
        for cp in copies:
            cp.wait()
        pltpu.sync_copy(rows_v, out_hbm.at[pl.ds(wid * rows_per_w, rows_per_w)])

    return sc_gather


def _rnn_step(x_ref, wih_ref, whh_ref, bias_ref, out_ref, h_ref):
    t = pl.program_id(0)

    @pl.when(t == 0)
    def _():
        h_ref[...] = jnp.zeros_like(h_ref)

    acc = jnp.dot(x_ref[0], wih_ref[...], preferred_element_type=jnp.float32)
    acc += jnp.dot(h_ref[...], whh_ref[...], preferred_element_type=jnp.float32)
    acc += bias_ref[...]
    h_new = jnp.tanh(acc)
    h_ref[...] = h_new
    out_ref[0] = h_new


def kernel(sentences, emb_table, W_ih, W_hh, b_ih, b_hh):
    batch, seq = sentences.shape
    vocab, emb = emb_table.shape
    hid = W_hh.shape[0]
    n_rows = batch * seq

    idx = sentences.T.astype(jnp.int32).reshape(n_rows // _CHUNK, _CHUNK)

    xg = _make_sc_gather(n_rows, emb, 32)(idx, emb_table)
    xg = xg.reshape(seq, batch, emb)

    wih_t = W_ih.T
    whh_t = W_hh.T
    bias = (b_ih + b_hh).reshape(1, hid)

    out_tm = pl.pallas_call(
        _rnn_step,
        grid=(seq,),
        in_specs=[
            pl.BlockSpec((1, batch, emb), lambda t: (t, 0, 0)),
            pl.BlockSpec((emb, hid), lambda t: (0, 0)),
            pl.BlockSpec((hid, hid), lambda t: (0, 0)),
            pl.BlockSpec((1, hid), lambda t: (0, 0)),
        ],
        out_specs=pl.BlockSpec((1, batch, hid), lambda t: (t, 0, 0)),
        out_shape=jax.ShapeDtypeStruct((seq, batch, hid), jnp.float32),
        scratch_shapes=[pltpu.VMEM((batch, hid), jnp.float32)],
    )(xg, wih_t, whh_t, bias)

    final_output = out_tm.transpose(1, 0, 2)
    h = out_tm[seq - 1][None]
    return final_output, h


# speedup vs baseline: 2.5074x; 2.5074x over previous
import functools

import jax
import jax.numpy as jnp
from jax import lax
from jax.experimental import pallas as pl
from jax.experimental.pallas import tpu as pltpu
from jax.experimental.pallas import tpu_sc as plsc

_CHUNK = 80


def _make_sc_gather(n_rows, width, n_workers):
    rows_per_w = n_rows // n_workers
    n_chunks = rows_per_w // _CHUNK
    assert rows_per_w % _CHUNK == 0

    mesh = plsc.VectorSubcoreMesh(core_axis_name="c", subcore_axis_name="s")

    @functools.partial(
        pl.kernel,
        mesh=mesh,
        out_type=jax.ShapeDtypeStruct((n_rows, width), jnp.float32),
        scratch_types=[
            pltpu.VMEM((rows_per_w,), jnp.int32),
            pltpu.VMEM((2, _CHUNK, width), jnp.float32),
            pltpu.SemaphoreType.DMA,
        ],
    )
    def sc_gather(idx_hbm, table_hbm, out_hbm, idx_v, rows_v, sem):
        nc = 2
        wid = lax.axis_index("s") * nc + lax.axis_index("c")
        base = wid * rows_per_w
        pltpu.sync_copy(idx_hbm.at[pl.ds(base, rows_per_w)], idx_v)

        def gather(j, buf):
            return pltpu.async_copy(
                table_hbm.at[idx_v.at[pl.ds(j * _CHUNK, _CHUNK)]],
                rows_v.at[buf],
                sem,
            )

        cp = gather(0, 0)
        for j in range(n_chunks):
            nxt = gather(j + 1, (j + 1) % 2) if j + 1 < n_chunks else None
            cp.wait()
            pltpu.sync_copy(
                rows_v.at[j % 2], out_hbm.at[pl.ds(base + j * _CHUNK, _CHUNK)]
            )
            cp = nxt

    return sc_gather


def _rnn_step(x_ref, par_ref, wih_ref, whh_ref, bias_ref, out_ref, h_ref):
    t = pl.program_id(0)

    @pl.when(t == 0)
    def _():
        h_ref[...] = jnp.zeros_like(h_ref)

    hid = h_ref.shape[-1]
    x128 = x_ref[0]
    seq = par_ref.shape[-1]
    onehot = (lax.broadcasted_iota(jnp.int32, (seq, 1), 0) == t).astype(jnp.float32)
    par = jnp.dot(par_ref[...], onehot, preferred_element_type=jnp.float32)
    x = jnp.where(par > 0.5, x128[:, hid:], x128[:, :hid])
    acc = jnp.dot(x, wih_ref[...], preferred_element_type=jnp.float32)
    acc += jnp.dot(h_ref[...], whh_ref[...], preferred_element_type=jnp.float32)
    acc += bias_ref[...]
    h_new = jnp.tanh(acc)
    h_ref[...] = h_new
    out_ref[0] = h_new


def kernel(sentences, emb_table, W_ih, W_hh, b_ih, b_hh):
    batch, seq = sentences.shape
    vocab, emb = emb_table.shape
    hid = W_hh.shape[0]
    n_rows = batch * seq

    table2 = emb_table.reshape(vocab // 2, 2 * emb)

    idx = sentences.T.astype(jnp.int32).reshape(n_rows)
    phys = idx >> 1
    parity = (sentences & 1).astype(jnp.float32)

    xg = _make_sc_gather(n_rows, 2 * emb, 32)(phys, table2)
    xg = xg.reshape(seq, batch, 2 * emb)

    wih_t = W_ih.T
    whh_t = W_hh.T
    bias = (b_ih + b_hh).reshape(1, hid)

    out_tm = pl.pallas_call(
        _rnn_step,
        grid=(seq,),
        in_specs=[
            pl.BlockSpec((1, batch, 2 * emb), lambda t: (t, 0, 0)),
            pl.BlockSpec((batch, seq), lambda t: (0, 0)),
            pl.BlockSpec((emb, hid), lambda t: (0, 0)),
            pl.BlockSpec((hid, hid), lambda t: (0, 0)),
            pl.BlockSpec((1, hid), lambda t: (0, 0)),
        ],
        out_specs=pl.BlockSpec((1, batch, hid), lambda t: (t, 0, 0)),
        out_shape=jax.ShapeDtypeStruct((seq, batch, hid), jnp.float32),
        scratch_shapes=[pltpu.VMEM((batch, hid), jnp.float32)],
    )(xg, parity, wih_t, whh_t, bias)

    final_output = out_tm.transpose(1, 0, 2)
    h = out_tm[seq - 1][None]
    return final_output, h
